# baseline (speedup 1.0000x reference)
import jax
import jax.numpy as jnp
from jax.experimental import pallas as pl
from jax.experimental.pallas import tpu as pltpu

_D = 512
_P = 8
_T = 128
_G = 4
_NG = _T // _G
_K = _G * _P
_N = _G * _D
_BB = 8


def _body(x_ref, ws_ref, brep_ref, pos_ref, o_ref):
    x = x_ref[...]
    brep = brep_ref[...]
    for g in range(_NG):
        xg = x[:, _K * g:_K * (g + 1)]
        yg = jax.lax.dot_general(
            xg, ws_ref[...],
            (((1,), (0,)), ((), ())),
            preferred_element_type=jnp.float32,
        )
        padd = pos_ref[0:1, _N * g:_N * (g + 1)]
        o_ref[:, _N * g:_N * (g + 1)] = yg + padd + brep


def kernel(inputs, W_proj, b_proj, pos_table):
    B = inputs.shape[0]
    W_stack = jax.scipy.linalg.block_diag(*([W_proj] * _G))
    b_rep = jnp.tile(b_proj, _G).reshape(1, _N)
    pos2d = pos_table[:_T].reshape(1, _T * _D)
    out2d = pl.pallas_call(
        _body,
        grid=(B // _BB,),
        in_specs=[
            pl.BlockSpec((_BB, _T * _P), lambda i: (i, 0)),
            pl.BlockSpec((_K, _N), lambda i: (0, 0)),
            pl.BlockSpec((1, _N), lambda i: (0, 0)),
            pl.BlockSpec((1, _T * _D), lambda i: (0, 0)),
        ],
        out_specs=pl.BlockSpec((_BB, _T * _D), lambda i: (i, 0)),
        out_shape=jax.ShapeDtypeStruct((B, _T * _D), jnp.float32),
        compiler_params=pltpu.CompilerParams(
            dimension_semantics=("arbitrary",),
        ),
    )(inputs, W_stack, b_rep, pos2d)
    return out2d.reshape(B, _T, _D)


# speedup vs baseline: 2.5067x; 2.5067x over previous
import jax
import jax.numpy as jnp
from jax.experimental import pallas as pl
from jax.experimental.pallas import tpu as pltpu

_D = 512
_P = 8
_T = 128
_BB = 8


def _body(xt_ref, w_ref, b_ref, pos_ref, o_ref):
    w = w_ref[...]
    add = pos_ref[...] + b_ref[...]
    for b in range(_BB):
        xb = xt_ref[b]
        yb = jax.lax.dot_general(
            xb, w,
            (((0,), (0,)), ((), ())),
            preferred_element_type=jnp.float32,
        )
        o_ref[b] = yb + add


def kernel(inputs, W_proj, b_proj, pos_table):
    B = inputs.shape[0]
    xt = jnp.swapaxes(inputs.reshape(B, _T, _P), 1, 2)
    b2 = b_proj.reshape(1, _D)
    return pl.pallas_call(
        _body,
        grid=(B // _BB,),
        in_specs=[
            pl.BlockSpec((_BB, _P, _T), lambda i: (i, 0, 0)),
            pl.BlockSpec((_P, _D), lambda i: (0, 0)),
            pl.BlockSpec((1, _D), lambda i: (0, 0)),
            pl.BlockSpec((_T, _D), lambda i: (0, 0)),
        ],
        out_specs=pl.BlockSpec((_BB, _T, _D), lambda i: (i, 0, 0)),
        out_shape=jax.ShapeDtypeStruct((B, _T, _D), jnp.float32),
        compiler_params=pltpu.CompilerParams(
            dimension_semantics=("arbitrary",),
        ),
    )(xt, W_proj, b2, pos_table)


# speedup vs baseline: 3.7429x; 1.4932x over previous
import jax
import jax.numpy as jnp
from jax.experimental import pallas as pl
from jax.experimental.pallas import tpu as pltpu

_D = 512
_P = 8
_T = 128
_BB = 32


def _body(xt_ref, w_ref, b_ref, pos_ref, o_ref):
    w = w_ref[...]
    add = pos_ref[...] + b_ref[...]
    for b in range(_BB):
        xb = xt_ref[b]
        yb = jax.lax.dot_general(
            xb, w,
            (((0,), (0,)), ((), ())),
            preferred_element_type=jnp.float32,
        )
        o_ref[b] = yb + add


def kernel(inputs, W_proj, b_proj, pos_table):
    B = inputs.shape[0]
    xt = jnp.swapaxes(inputs.reshape(B, _T, _P), 1, 2)
    b2 = b_proj.reshape(1, _D)
    return pl.pallas_call(
        _body,
        grid=(B // _BB,),
        in_specs=[
            pl.BlockSpec((_BB, _P, _T), lambda i: (i, 0, 0)),
            pl.BlockSpec((_P, _D), lambda i: (0, 0)),
            pl.BlockSpec((1, _D), lambda i: (0, 0)),
            pl.BlockSpec((_T, _D), lambda i: (0, 0)),
        ],
        out_specs=pl.BlockSpec((_BB, _T, _D), lambda i: (i, 0, 0)),
        out_shape=jax.ShapeDtypeStruct((B, _T, _D), jnp.float32),
        compiler_params=pltpu.CompilerParams(
            dimension_semantics=("arbitrary",),
        ),
    )(xt, W_proj, b2, pos_table)
